# baseline (device time: 15049 ns/iter reference)
import jax
import jax.numpy as jnp
from jax import lax
from jax.experimental import pallas as pl
from jax.experimental.pallas import tpu as pltpu

N_DEV = 32

_sem_signal = getattr(pl, "semaphore_signal", None) or pltpu.semaphore_signal
_sem_wait = getattr(pl, "semaphore_wait", None) or pltpu.semaphore_wait
_CompilerParams = getattr(pltpu, "CompilerParams", None) or pltpu.TPUCompilerParams
_DeviceIdType = getattr(pl, "DeviceIdType", None) or pltpu.DeviceIdType

_SEND_ORDER = sorted(range(1, N_DEV), key=lambda d: -min(d, N_DEV - d))
_RECV_ORDER = sorted(range(1, N_DEV), key=lambda d: min(d, N_DEV - d))


def kernel(x):
    m_per, n = x.shape

    n_chunks = 4
    chunk = m_per // n_chunks
    assert chunk * n_chunks == m_per

    def body(x_hbm, out_ref, xv_ref, comm_ref, send_sems, recv_sems, load_sems):
        my_pos = lax.axis_index("i")

        def chunk_copy(c):
            return pltpu.make_async_copy(
                x_hbm.at[pl.ds(c * chunk, chunk), :],
                xv_ref.at[c % 2],
                load_sems.at[c % 2],
            )

        copies = [chunk_copy(c) for c in range(n_chunks)]
        copies[0].start()
        copies[1].start()

        barrier_sem = pltpu.get_barrier_semaphore()
        _sem_signal(barrier_sem, inc=1)
        _sem_wait(barrier_sem, 1)

        acc = None
        for c in range(n_chunks):
            copies[c].wait()
            part = jnp.max(xv_ref[c % 2], axis=0, keepdims=True)
            acc = part if acc is None else jnp.maximum(acc, part)
            if c + 2 < n_chunks:
                copies[c + 2].start()
        comm_ref[pl.ds(my_pos, 1), :] = acc

        sends = []
        for d in _SEND_ORDER:
            s = pltpu.make_async_remote_copy(
                src_ref=comm_ref.at[my_pos],
                dst_ref=comm_ref.at[my_pos],
                send_sem=send_sems.at[d],
                recv_sem=recv_sems.at[my_pos],
                device_id=((my_pos + d) % N_DEV,),
                device_id_type=_DeviceIdType.MESH,
            )
            s.start()
            sends.append(s)

        for d in _RECV_ORDER:
            src_pos = (my_pos + d) % N_DEV
            recv = pltpu.make_async_remote_copy(
                src_ref=comm_ref.at[src_pos],
                dst_ref=comm_ref.at[src_pos],
                send_sem=send_sems.at[d],
                recv_sem=recv_sems.at[src_pos],
                device_id=(my_pos,),
                device_id_type=_DeviceIdType.MESH,
            )
            recv.wait_recv()

        out_ref[:, :] = jnp.max(comm_ref[:, :], axis=0, keepdims=True)
        for s in sends:
            s.wait_send()

    return pl.pallas_call(
        body,
        out_shape=jax.ShapeDtypeStruct((1, n), x.dtype),
        in_specs=[pl.BlockSpec(memory_space=pl.ANY)],
        out_specs=pl.BlockSpec(memory_space=pltpu.VMEM),
        scratch_shapes=[
            pltpu.VMEM((2, m_per // n_chunks, n), x.dtype),
            pltpu.VMEM((N_DEV, n), x.dtype),
            pltpu.SemaphoreType.DMA((N_DEV,)),
            pltpu.SemaphoreType.DMA((N_DEV,)),
            pltpu.SemaphoreType.DMA((2,)),
        ],
        compiler_params=_CompilerParams(collective_id=0),
    )(x)


# device time: 14379 ns/iter; 1.0466x vs baseline; 1.0466x over previous
import jax
import jax.numpy as jnp
from jax import lax
from jax.experimental import pallas as pl
from jax.experimental.pallas import tpu as pltpu

N_DEV = 32

_sem_signal = getattr(pl, "semaphore_signal", None) or pltpu.semaphore_signal
_sem_wait = getattr(pl, "semaphore_wait", None) or pltpu.semaphore_wait
_CompilerParams = getattr(pltpu, "CompilerParams", None) or pltpu.TPUCompilerParams
_DeviceIdType = getattr(pl, "DeviceIdType", None) or pltpu.DeviceIdType

_SEND_ORDER = sorted(range(1, N_DEV), key=lambda d: -min(d, N_DEV - d))
_RECV_ORDER = sorted(range(1, N_DEV), key=lambda d: min(d, N_DEV - d))


def kernel(x):
    m_per, n = x.shape

    def body(x_hbm, out_ref, xv_ref, comm_ref, send_sems, recv_sems, load_sem):
        my_pos = lax.axis_index("i")

        load = pltpu.make_async_copy(x_hbm, xv_ref, load_sem)
        load.start()

        barrier_sem = pltpu.get_barrier_semaphore()
        _sem_signal(barrier_sem, inc=1)
        _sem_wait(barrier_sem, 1)

        load.wait()
        comm_ref[pl.ds(my_pos, 1), :] = jnp.max(
            xv_ref[:, :], axis=0, keepdims=True
        )

        sends = []
        for d in _SEND_ORDER:
            s = pltpu.make_async_remote_copy(
                src_ref=comm_ref.at[my_pos],
                dst_ref=comm_ref.at[my_pos],
                send_sem=send_sems.at[d],
                recv_sem=recv_sems.at[my_pos],
                device_id=((my_pos + d) % N_DEV,),
                device_id_type=_DeviceIdType.MESH,
            )
            s.start()
            sends.append(s)

        for d in _RECV_ORDER:
            src_pos = (my_pos + d) % N_DEV
            recv = pltpu.make_async_remote_copy(
                src_ref=comm_ref.at[src_pos],
                dst_ref=comm_ref.at[src_pos],
                send_sem=send_sems.at[d],
                recv_sem=recv_sems.at[src_pos],
                device_id=(my_pos,),
                device_id_type=_DeviceIdType.MESH,
            )
            recv.wait_recv()

        out_ref[:, :] = jnp.max(comm_ref[:, :], axis=0, keepdims=True)
        for s in sends:
            s.wait_send()

    return pl.pallas_call(
        body,
        out_shape=jax.ShapeDtypeStruct((1, n), x.dtype),
        in_specs=[pl.BlockSpec(memory_space=pl.ANY)],
        out_specs=pl.BlockSpec(memory_space=pltpu.VMEM),
        scratch_shapes=[
            pltpu.VMEM((m_per, n), x.dtype),
            pltpu.VMEM((N_DEV, n), x.dtype),
            pltpu.SemaphoreType.DMA((N_DEV,)),
            pltpu.SemaphoreType.DMA((N_DEV,)),
            pltpu.SemaphoreType.DMA,
        ],
        compiler_params=_CompilerParams(collective_id=0),
    )(x)
